# baseline (speedup 1.0000x reference)
import functools

import jax
import jax.numpy as jnp
from jax import lax
from jax.experimental import pallas as pl
from jax.experimental.pallas import tpu as pltpu
from jax.experimental.pallas import tpu_sc as plsc

EPS = 1e-5
NC, NS, LANES = 2, 16, 16
CH = 128
FB = 64

_SC_PARAMS = pltpu.CompilerParams(use_tc_tiling_on_sc=False)
_TC_PARAMS = pltpu.CompilerParams(vmem_limit_bytes=100 * 1024 * 1024)


def _mesh():
    return plsc.VectorSubcoreMesh(
        core_axis_name="c", subcore_axis_name="s", num_cores=NC, num_subcores=NS
    )


def _make_deg(NP, CD):
    TPB = NP // NS
    DR = NP // 4

    @functools.partial(
        pl.kernel,
        mesh=_mesh(),
        out_type=jax.ShapeDtypeStruct((NC, NP, LANES), jnp.float32),
        scratch_types=[
            pltpu.VMEM((CD, CH), jnp.int32),
            pltpu.VMEM((CH, LANES), jnp.float32),
            pltpu.VMEM((TPB, LANES), jnp.float32),
            pltpu.VMEM((DR, LANES), jnp.float32),
            pltpu.VMEM_SHARED((NP, LANES), jnp.float32),
        ],
        compiler_params=_SC_PARAMS,
    )
    def deg_k(dst_hbm, out_hbm, dst_v, ones_v, zer_v, buf_v, acc):
        c = lax.axis_index("c")
        s = lax.axis_index("s")
        wid = c * NS + s
        one = jnp.full((LANES,), 1.0, jnp.float32)
        zero = jnp.zeros((LANES,), jnp.float32)

        @pl.loop(0, CH)
        def _(i):
            ones_v[i, :] = one

        @pl.loop(0, TPB)
        def _(i):
            zer_v[i, :] = zero

        pltpu.sync_copy(zer_v, acc.at[pl.ds(s * TPB, TPB)])
        pltpu.sync_copy(dst_hbm.at[wid], dst_v)
        plsc.subcore_barrier()

        @pl.loop(0, CD)
        def _(j):
            pltpu.sync_copy(ones_v, acc.at[dst_v.at[j]], add=True)

        plsc.subcore_barrier()

        @pl.when(s < 4)
        def _():
            pltpu.sync_copy(acc.at[pl.ds(s * DR, DR)], buf_v)
            pltpu.sync_copy(buf_v, out_hbm.at[c, pl.ds(s * DR, DR)])

    return deg_k


def _make_agg(NP, CT, NPH):
    TPB = NP // NS
    NFULL = TPB // CH
    REM = TPB - NFULL * CH
    NB = 2

    @functools.partial(
        pl.kernel,
        mesh=_mesh(),
        out_type=jax.ShapeDtypeStruct((NPH * NC, NP, FB), jnp.float32),
        scratch_types=[
            pltpu.VMEM((CT, CH), jnp.int32),
            pltpu.VMEM((CT, CH), jnp.int32),
            pltpu.VMEM((NB, CH, FB), jnp.float32),
            pltpu.VMEM((CH, FB), jnp.float32),
            [pltpu.SemaphoreType.DMA] * NB,
            pltpu.VMEM_SHARED((NP, FB), jnp.float32),
        ],
        compiler_params=_SC_PARAMS,
    )
    def agg_k(table_hbm, src_hbm, dst_hbm, out_hbm,
              src_v, dst_v, rows_v, buf_v, gsems, acc):
        c = lax.axis_index("c")
        s = lax.axis_index("s")
        base = s * TPB

        pltpu.sync_copy(dst_hbm.at[c, s], dst_v)

        for p in range(NPH):
            trow = (p * NC + c) * NP + base
            for q in range(NFULL):
                pltpu.sync_copy(table_hbm.at[pl.ds(trow + q * CH, CH)], buf_v)
                pltpu.sync_copy(buf_v, acc.at[pl.ds(base + q * CH, CH)])
            pltpu.sync_copy(table_hbm.at[pl.ds(trow + NFULL * CH, REM)],
                            buf_v.at[pl.ds(0, REM)])
            pltpu.sync_copy(buf_v.at[pl.ds(0, REM)],
                            acc.at[pl.ds(base + NFULL * CH, REM)])
            pltpu.sync_copy(src_hbm.at[p, c, s], src_v)
            plsc.subcore_barrier()

            for b in range(2):
                pltpu.async_copy(table_hbm.at[src_v.at[b]], rows_v.at[b],
                                 gsems[b])

            @pl.loop(0, CT, step=2)
            def _(g):
                for b in range(2):
                    j = g + b
                    pltpu.make_async_copy(
                        table_hbm.at[src_v.at[j]], rows_v.at[b],
                        gsems[b]).wait()
                    pltpu.sync_copy(rows_v.at[b], acc.at[dst_v.at[j]],
                                    add=True)
                    jn = j + 2

                    @pl.when(jn < CT)
                    def _():
                        pltpu.async_copy(
                            table_hbm.at[src_v.at[jn]], rows_v.at[b],
                            gsems[b])

            plsc.subcore_barrier()
            blk = p * NC + c
            for q in range(NFULL):
                pltpu.sync_copy(acc.at[pl.ds(base + q * CH, CH)], buf_v)
                pltpu.sync_copy(buf_v,
                                out_hbm.at[blk, pl.ds(base + q * CH, CH)])
            pltpu.sync_copy(acc.at[pl.ds(base + NFULL * CH, REM)],
                            buf_v.at[pl.ds(0, REM)])
            pltpu.sync_copy(buf_v.at[pl.ds(0, REM)],
                            out_hbm.at[blk, pl.ds(base + NFULL * CH, REM)])

    return agg_k


def _kb(NP, N):
    def body(deg_ref, x_ref, w1_ref, dinv_ref, t1_ref):
        deg = deg_ref[0] + deg_ref[1] + 1.0
        dinv_all = lax.rsqrt(deg)
        dinv_ref[...] = dinv_all[:, 0:1]
        dinv = dinv_all[0:N, 0:1]
        t1 = jnp.dot(x_ref[...], w1_ref[...],
                     preferred_element_type=jnp.float32) * dinv
        for b in range(NC):
            t1_ref[b, 0:N, :] = t1[:, b * FB:(b + 1) * FB]
            t1_ref[b, N:NP, :] = jnp.zeros((NP - N, FB), jnp.float32)
    return body


def _kd(NP, N):
    def body(agg_ref, dinv_ref, b1_ref, g1_ref, be1_ref, w2_ref, t2_ref):
        dinv = dinv_ref[0:N, :]
        t2 = None
        for b in range(NC):
            sarr = agg_ref[b, 0:N, :] * dinv + b1_ref[b]
            h = jnp.maximum(sarr, 0.0)
            mu = jnp.mean(h, axis=0, keepdims=True)
            var = jnp.mean((h - mu) ** 2, axis=0, keepdims=True)
            hn = (h - mu) * lax.rsqrt(var + EPS) * g1_ref[b] + be1_ref[b]
            d = jnp.dot(hn, w2_ref[b], preferred_element_type=jnp.float32)
            t2 = d if t2 is None else t2 + d
        t2 = t2 * dinv
        for b in range(4):
            t2_ref[b, 0:N, :] = t2[:, b * FB:(b + 1) * FB]
            t2_ref[b, N:NP, :] = jnp.zeros((NP - N, FB), jnp.float32)
    return body


def _kf(NP, N, B):
    def body(agg_ref, dinv_ref, b2_ref, g2_ref, be2_ref, batch_ref,
             rdkit_ref, w3e_ref, w3r_ref, b3_ref, g3_ref, be3_ref, w4_ref,
             b4_ref, g4_ref, be4_ref, w5_ref, b5_ref, out_ref):
        dinv = dinv_ref[0:N, :]
        onehot = (batch_ref[...] ==
                  lax.broadcasted_iota(jnp.int32, (B, N), 0)).astype(jnp.float32)
        counts = jnp.dot(onehot, jnp.ones((N, 1), jnp.float32),
                         preferred_element_type=jnp.float32)
        inv = 1.0 / jnp.maximum(counts, 1.0)
        z = jnp.dot(rdkit_ref[...], w3r_ref[...],
                    preferred_element_type=jnp.float32)
        for b in range(4):
            sarr = agg_ref[b, 0:N, :] * dinv + b2_ref[b]
            h = jnp.maximum(sarr, 0.0)
            mu = jnp.mean(h, axis=0, keepdims=True)
            var = jnp.mean((h - mu) ** 2, axis=0, keepdims=True)
            seg = jnp.dot(onehot, h, preferred_element_type=jnp.float32)
            emb = ((seg * inv - mu) * lax.rsqrt(var + EPS) * g2_ref[b]
                   + be2_ref[b])
            z = z + jnp.dot(emb, w3e_ref[b],
                            preferred_element_type=jnp.float32)
        z = jnp.maximum(z + b3_ref[...], 0.0)
        mu = jnp.mean(z, axis=0, keepdims=True)
        var = jnp.mean((z - mu) ** 2, axis=0, keepdims=True)
        z = (z - mu) * lax.rsqrt(var + EPS) * g3_ref[...] + be3_ref[...]
        z = jnp.maximum(
            jnp.dot(z, w4_ref[...], preferred_element_type=jnp.float32)
            + b4_ref[...], 0.0)
        mu = jnp.mean(z, axis=0, keepdims=True)
        var = jnp.mean((z - mu) ** 2, axis=0, keepdims=True)
        z = (z - mu) * lax.rsqrt(var + EPS) * g4_ref[...] + be4_ref[...]
        out_ref[...] = (jnp.dot(z, w5_ref[...],
                                preferred_element_type=jnp.float32)
                        + b5_ref[...])
    return body


def kernel(x, edge_index, batch, rdkit_feats, W1, b1, gamma1, beta1, W2, b2,
           gamma2, beta2, W3, b3, gamma3, beta3, W4, b4, gamma4, beta4, W5,
           b5):
    N, D = x.shape
    E = edge_index.shape[1]
    B, R = rdkit_feats.shape
    H = W1.shape[1]
    NP = N + 16

    CT = -(-E // (NS * CH))
    CT = CT + (-CT) % 4
    EP = NS * CH * CT
    CD = EP // (NC * NS * CH)

    src = edge_index[0].astype(jnp.int32)
    dst = edge_index[1].astype(jnp.int32)
    padv = jnp.full((EP - E,), N, jnp.int32)
    src_p = jnp.concatenate([src, padv])
    dst_p = jnp.concatenate([dst, padv])
    offs1 = jnp.arange(NC, dtype=jnp.int32) * NP
    offs2 = jnp.arange(2 * NC, dtype=jnp.int32) * NP
    src1 = (src_p[None, :] + offs1[:, None]).reshape(1, NC, NS, CT, CH)
    src2 = (src_p[None, :] + offs2[:, None]).reshape(2, NC, NS, CT, CH)
    dst_c = jnp.broadcast_to(dst_p, (NC, EP)).reshape(NC, NS, CT, CH)
    dst_deg = dst_p.reshape(NC * NS, CD, CH)
    batchT = batch.astype(jnp.int32).reshape(1, N)

    deg_parts = _make_deg(NP, CD)(dst_deg)

    dinv, t1 = pl.pallas_call(
        _kb(NP, N),
        out_shape=(
            jax.ShapeDtypeStruct((NP, 1), jnp.float32),
            jax.ShapeDtypeStruct((NC, NP, FB), jnp.float32),
        ),
        compiler_params=_TC_PARAMS,
    )(deg_parts, x, W1)

    agg1 = _make_agg(NP, CT, 1)(t1.reshape(NC * NP, FB), src1, dst_c)

    t2 = pl.pallas_call(
        _kd(NP, N),
        out_shape=jax.ShapeDtypeStruct((4, NP, FB), jnp.float32),
        compiler_params=_TC_PARAMS,
    )(agg1, dinv, b1.reshape(NC, 1, FB), gamma1.reshape(NC, 1, FB),
      beta1.reshape(NC, 1, FB), W2.reshape(NC, FB, 2 * H))

    agg2 = _make_agg(NP, CT, 2)(t2.reshape(4 * NP, FB), src2, dst_c)

    out = pl.pallas_call(
        _kf(NP, N, B),
        out_shape=jax.ShapeDtypeStruct((B, 1), jnp.float32),
        compiler_params=_TC_PARAMS,
    )(agg2, dinv, b2.reshape(4, 1, FB), gamma2.reshape(4, 1, FB),
      beta2.reshape(4, 1, FB), batchT, rdkit_feats,
      W3[:B].reshape(4, FB, 2 * H), W3[B:], b3.reshape(1, 2 * H),
      gamma3.reshape(1, 2 * H), beta3.reshape(1, 2 * H), W4,
      b4.reshape(1, H), gamma4.reshape(1, H), beta4.reshape(1, H), W5,
      b5.reshape(1, 1))

    return out[:, 0]


# speedup vs baseline: 1.4411x; 1.4411x over previous
import functools

import jax
import jax.numpy as jnp
from jax import lax
from jax.experimental import pallas as pl
from jax.experimental.pallas import tpu as pltpu
from jax.experimental.pallas import tpu_sc as plsc

EPS = 1e-5
NC, NS, LANES = 2, 16, 16
CH = 128
FB = 64

_SC_PARAMS = pltpu.CompilerParams(use_tc_tiling_on_sc=False)
_TC_PARAMS = pltpu.CompilerParams(vmem_limit_bytes=100 * 1024 * 1024)


def _mesh():
    return plsc.VectorSubcoreMesh(
        core_axis_name="c", subcore_axis_name="s", num_cores=NC, num_subcores=NS
    )


def _make_deg(NP, CD):
    TPB = NP // NS
    DR = NP // 4

    @functools.partial(
        pl.kernel,
        mesh=_mesh(),
        out_type=jax.ShapeDtypeStruct((NC, NP, LANES), jnp.float32),
        scratch_types=[
            pltpu.VMEM((CD, CH), jnp.int32),
            pltpu.VMEM((CH, LANES), jnp.float32),
            pltpu.VMEM((TPB, LANES), jnp.float32),
            pltpu.VMEM((DR, LANES), jnp.float32),
            pltpu.VMEM_SHARED((NP, LANES), jnp.float32),
        ],
        compiler_params=_SC_PARAMS,
    )
    def deg_k(dst_hbm, out_hbm, dst_v, ones_v, zer_v, buf_v, acc):
        c = lax.axis_index("c")
        s = lax.axis_index("s")
        wid = c * NS + s
        one = jnp.full((LANES,), 1.0, jnp.float32)
        zero = jnp.zeros((LANES,), jnp.float32)

        @pl.loop(0, CH)
        def _(i):
            ones_v[i, :] = one

        @pl.loop(0, TPB)
        def _(i):
            zer_v[i, :] = zero

        pltpu.sync_copy(zer_v, acc.at[pl.ds(s * TPB, TPB)])
        pltpu.sync_copy(dst_hbm.at[wid], dst_v)
        plsc.subcore_barrier()

        @pl.loop(0, CD)
        def _(j):
            pltpu.sync_copy(ones_v, acc.at[dst_v.at[j]], add=True)

        plsc.subcore_barrier()

        @pl.when(s < 4)
        def _():
            pltpu.sync_copy(acc.at[pl.ds(s * DR, DR)], buf_v)
            pltpu.sync_copy(buf_v, out_hbm.at[c, pl.ds(s * DR, DR)])

    return deg_k


def _make_agg(NP, CT, NPH):
    TPB = NP // NS
    NFULL = TPB // CH
    REM = TPB - NFULL * CH

    @functools.partial(
        pl.kernel,
        mesh=_mesh(),
        out_type=jax.ShapeDtypeStruct((NPH, NP, NC * FB), jnp.float32),
        scratch_types=[
            pltpu.VMEM((CT, CH), jnp.int32),
            pltpu.VMEM((CT, CH), jnp.int32),
            pltpu.VMEM((2, CH, FB), jnp.float32),
            pltpu.VMEM((CH, FB), jnp.float32),
            pltpu.SemaphoreType.DMA,
            pltpu.SemaphoreType.DMA,
            pltpu.VMEM_SHARED((NP, FB), jnp.float32),
        ],
        compiler_params=_SC_PARAMS,
    )
    def agg_k(table_hbm, src_hbm, dst_hbm, out_hbm,
              src_v, dst_v, rows_v, buf_v, sem0, sem1, acc):
        c = lax.axis_index("c")
        s = lax.axis_index("s")
        base = s * TPB
        sems = (sem0, sem1)

        pltpu.sync_copy(dst_hbm.at[c, s], dst_v)

        for p in range(NPH):
            blk = p * NC + c
            trow = blk * NP + base
            for q in range(NFULL):
                pltpu.sync_copy(table_hbm.at[pl.ds(trow + q * CH, CH)], buf_v)
                pltpu.sync_copy(buf_v, acc.at[pl.ds(base + q * CH, CH)])
            pltpu.sync_copy(table_hbm.at[pl.ds(trow + NFULL * CH, REM)],
                            buf_v.at[pl.ds(0, REM)])
            pltpu.sync_copy(buf_v.at[pl.ds(0, REM)],
                            acc.at[pl.ds(base + NFULL * CH, REM)])
            pltpu.sync_copy(src_hbm.at[p, c, s], src_v)
            plsc.subcore_barrier()

            pltpu.async_copy(table_hbm.at[src_v.at[0]], rows_v.at[0], sem0)
            pltpu.async_copy(table_hbm.at[src_v.at[1]], rows_v.at[1], sem1)

            @pl.loop(0, CT, step=2)
            def _(g):
                for b in range(2):
                    j = g + b
                    pltpu.make_async_copy(
                        table_hbm.at[src_v.at[j]], rows_v.at[b],
                        sems[b]).wait()
                    pltpu.sync_copy(rows_v.at[b], acc.at[dst_v.at[j]],
                                    add=True)
                    jn = j + 2

                    @pl.when(jn < CT)
                    def _():
                        pltpu.async_copy(
                            table_hbm.at[src_v.at[jn]], rows_v.at[b], sems[b])

            plsc.subcore_barrier()
            for q in range(NFULL):
                pltpu.sync_copy(acc.at[pl.ds(base + q * CH, CH)], buf_v)
                pltpu.sync_copy(buf_v, out_hbm.at[p, pl.ds(base + q * CH, CH),
                                                  pl.ds(c * FB, FB)])
            pltpu.sync_copy(acc.at[pl.ds(base + NFULL * CH, REM)],
                            buf_v.at[pl.ds(0, REM)])
            pltpu.sync_copy(buf_v.at[pl.ds(0, REM)],
                            out_hbm.at[p, pl.ds(base + NFULL * CH, REM),
                                       pl.ds(c * FB, FB)])

    return agg_k


def _kb(NP, N):
    def body(deg_ref, x_ref, w1_ref, dinv_ref, t1_ref):
        deg = deg_ref[0] + deg_ref[1] + 1.0
        dinv_all = lax.rsqrt(deg)
        dinv_ref[...] = dinv_all[:, 0:1]
        dinv = dinv_all[0:N, 0:1]
        t1 = jnp.dot(x_ref[...], w1_ref[...],
                     preferred_element_type=jnp.float32) * dinv
        for b in range(NC):
            t1_ref[b, 0:N, :] = t1[:, b * FB:(b + 1) * FB]
            t1_ref[b, N:NP, :] = jnp.zeros((NP - N, FB), jnp.float32)
    return body


def _kd(NP, N):
    def body(agg_ref, dinv_ref, b1_ref, g1_ref, be1_ref, w2_ref, t2_ref):
        dinv = dinv_ref[0:N, :]
        sarr = agg_ref[0, 0:N, :] * dinv + b1_ref[...]
        h = jnp.maximum(sarr, 0.0)
        mu = jnp.mean(h, axis=0, keepdims=True)
        var = jnp.mean((h - mu) ** 2, axis=0, keepdims=True)
        hn = (h - mu) * lax.rsqrt(var + EPS) * g1_ref[...] + be1_ref[...]
        t2 = jnp.dot(hn, w2_ref[...], preferred_element_type=jnp.float32)
        t2 = t2 * dinv
        for b in range(4):
            t2_ref[b, 0:N, :] = t2[:, b * FB:(b + 1) * FB]
            t2_ref[b, N:NP, :] = jnp.zeros((NP - N, FB), jnp.float32)
    return body


def _kf(NP, N, B):
    def body(agg_ref, dinv_ref, b2_ref, g2_ref, be2_ref, batch_ref,
             rdkit_ref, w3e_ref, w3r_ref, b3_ref, g3_ref, be3_ref, w4_ref,
             b4_ref, g4_ref, be4_ref, w5_ref, b5_ref, out_ref):
        dinv = dinv_ref[0:N, :]
        onehot = (batch_ref[...] ==
                  lax.broadcasted_iota(jnp.int32, (B, N), 0)).astype(jnp.float32)
        counts = jnp.dot(onehot, jnp.ones((N, 1), jnp.float32),
                         preferred_element_type=jnp.float32)
        inv = 1.0 / jnp.maximum(counts, 1.0)
        z = jnp.dot(rdkit_ref[...], w3r_ref[...],
                    preferred_element_type=jnp.float32)
        for p in range(NC):
            sarr = agg_ref[p, 0:N, :] * dinv + b2_ref[p]
            h = jnp.maximum(sarr, 0.0)
            mu = jnp.mean(h, axis=0, keepdims=True)
            var = jnp.mean((h - mu) ** 2, axis=0, keepdims=True)
            seg = jnp.dot(onehot, h, preferred_element_type=jnp.float32)
            emb = ((seg * inv - mu) * lax.rsqrt(var + EPS) * g2_ref[p]
                   + be2_ref[p])
            z = z + jnp.dot(emb, w3e_ref[p],
                            preferred_element_type=jnp.float32)
        z = jnp.maximum(z + b3_ref[...], 0.0)
        mu = jnp.mean(z, axis=0, keepdims=True)
        var = jnp.mean((z - mu) ** 2, axis=0, keepdims=True)
        z = (z - mu) * lax.rsqrt(var + EPS) * g3_ref[...] + be3_ref[...]
        z = jnp.maximum(
            jnp.dot(z, w4_ref[...], preferred_element_type=jnp.float32)
            + b4_ref[...], 0.0)
        mu = jnp.mean(z, axis=0, keepdims=True)
        var = jnp.mean((z - mu) ** 2, axis=0, keepdims=True)
        z = (z - mu) * lax.rsqrt(var + EPS) * g4_ref[...] + be4_ref[...]
        out_ref[...] = (jnp.dot(z, w5_ref[...],
                                preferred_element_type=jnp.float32)
                        + b5_ref[...])
    return body


def kernel(x, edge_index, batch, rdkit_feats, W1, b1, gamma1, beta1, W2, b2,
           gamma2, beta2, W3, b3, gamma3, beta3, W4, b4, gamma4, beta4, W5,
           b5):
    N, D = x.shape
    E = edge_index.shape[1]
    B, R = rdkit_feats.shape
    H = W1.shape[1]
    NP = N + 16

    CT = -(-E // (NS * CH))
    CT = CT + (CT % 2)
    EP = NS * CH * CT
    CD = EP // (NC * NS * CH)

    src = edge_index[0].astype(jnp.int32)
    dst = edge_index[1].astype(jnp.int32)
    padv = jnp.full((EP - E,), N, jnp.int32)
    src_p = jnp.concatenate([src, padv])
    dst_p = jnp.concatenate([dst, padv])
    offs1 = jnp.arange(NC, dtype=jnp.int32) * NP
    offs2 = jnp.arange(2 * NC, dtype=jnp.int32) * NP
    src1 = (src_p[None, :] + offs1[:, None]).reshape(1, NC, NS, CT, CH)
    src2 = (src_p[None, :] + offs2[:, None]).reshape(2, NC, NS, CT, CH)
    dst_b = jnp.broadcast_to(dst_p, (NC, EP)).reshape(NC, NS, CT, CH)
    dst_deg = dst_p.reshape(NC * NS, CD, CH)
    batchT = batch.astype(jnp.int32).reshape(1, N)

    deg_parts = _make_deg(NP, CD)(dst_deg)

    dinv, t1 = pl.pallas_call(
        _kb(NP, N),
        out_shape=(
            jax.ShapeDtypeStruct((NP, 1), jnp.float32),
            jax.ShapeDtypeStruct((NC, NP, FB), jnp.float32),
        ),
        compiler_params=_TC_PARAMS,
    )(deg_parts, x, W1)

    agg1 = _make_agg(NP, CT, 1)(t1.reshape(NC * NP, FB), src1, dst_b)

    t2 = pl.pallas_call(
        _kd(NP, N),
        out_shape=jax.ShapeDtypeStruct((4, NP, FB), jnp.float32),
        compiler_params=_TC_PARAMS,
    )(agg1, dinv, b1.reshape(1, H), gamma1.reshape(1, H),
      beta1.reshape(1, H), W2)

    agg2 = _make_agg(NP, CT, 2)(t2.reshape(4 * NP, FB), src2, dst_b)

    out = pl.pallas_call(
        _kf(NP, N, B),
        out_shape=jax.ShapeDtypeStruct((B, 1), jnp.float32),
        compiler_params=_TC_PARAMS,
    )(agg2, dinv, b2.reshape(NC, 1, 2 * FB), gamma2.reshape(NC, 1, 2 * FB),
      beta2.reshape(NC, 1, 2 * FB), batchT, rdkit_feats,
      W3[:B].reshape(NC, 2 * FB, 2 * H), W3[B:], b3.reshape(1, 2 * H),
      gamma3.reshape(1, 2 * H), beta3.reshape(1, 2 * H), W4,
      b4.reshape(1, H), gamma4.reshape(1, H), beta4.reshape(1, H), W5,
      b5.reshape(1, 1))

    return out[:, 0]
